# baseline (speedup 1.0000x reference)
import jax
import jax.numpy as jnp
from jax import lax
from jax.experimental import pallas as pl
from jax.experimental.pallas import tpu as pltpu
from jax.experimental.pallas import tpu_sc as plsc

N_NODES = 10000
DIM = 128
N_EDGES = 320000
BATCH = 4096
N_LAYERS = 2

NUM_CORES = 2
NUM_SUBCORES = 16
NW = NUM_CORES * NUM_SUBCORES

CHUNK = 64
CHUNKS_PER_TILE = 160
E_PAD = NW * CHUNKS_PER_TILE * CHUNK // NUM_CORES
NBUF = 2

STRIPE = 632
LAST_STRIPE = N_NODES - (NUM_SUBCORES - 1) * STRIPE

_MESH = plsc.VectorSubcoreMesh(core_axis_name="c", subcore_axis_name="s")


def _layer_body(table_hbm, src_hbm, dst_hbm, w_hbm, part_hbm,
                src_v, dst_v, w_v, rows_v, acc_sh, gsems, ssems):
    core = lax.axis_index("c")
    sub = lax.axis_index("s")
    wid = core * NUM_SUBCORES + sub

    zvec = jnp.zeros((16,), jnp.float32)

    scope_zero = jax.named_scope("ph_zero")
    scope_zero.__enter__()

    def zfill(r, _):
        for v in range(DIM // 16):
            rows_v[0, r, pl.ds(v * 16, 16)] = zvec
        return 0

    lax.fori_loop(0, CHUNK, zfill, 0)

    def zcopy(i, _):
        off = jnp.minimum(sub * STRIPE + CHUNK * i, N_NODES - CHUNK)
        pltpu.sync_copy(rows_v.at[0], acc_sh.at[pl.ds(off, CHUNK)])
        return 0

    lax.fori_loop(0, STRIPE // CHUNK + 1, zcopy, 0)
    scope_zero.__exit__(None, None, None)

    plsc.subcore_barrier()

    def _scale(b, c):
        def scale_body(g, _):
            wvec = w_v[c, pl.ds(g * 16, 16)]
            for j in range(16):
                ws = wvec[j]
                for v in range(DIM // 16):
                    sl = pl.ds(v * 16, 16)
                    rows_v[b, g * 16 + j, sl] = rows_v[b, g * 16 + j, sl] * ws
            return 0

        lax.fori_loop(0, CHUNK // 16, scale_body, 0)

    seg_chunks = CHUNKS_PER_TILE // 10

    def _gather(c, i):
        pltpu.async_copy(table_hbm.at[src_v.at[c]], rows_v.at[i], gsems.at[i])

    def _gather_wait(c, i):
        pltpu.make_async_copy(table_hbm.at[src_v.at[c]], rows_v.at[i],
                              gsems.at[i]).wait()

    def seg_body(seg, _):
        base_row = wid * CHUNKS_PER_TILE + seg * seg_chunks
        pltpu.sync_copy(src_hbm.at[pl.ds(base_row, seg_chunks)], src_v)
        pltpu.sync_copy(dst_hbm.at[pl.ds(base_row, seg_chunks)], dst_v)
        pltpu.sync_copy(w_hbm.at[pl.ds(base_row, seg_chunks)], w_v)

        for i in range(NBUF):
            _gather(i, i)

        def step_body(q, _):
            cb = q * NBUF
            for i in range(NBUF):
                _gather_wait(cb + i, i)
                _scale(i, cb + i)
                pltpu.async_copy(rows_v.at[i], acc_sh.at[dst_v.at[cb + i]],
                                 ssems.at[i], add=True)
            for i in range(NBUF):
                pltpu.make_async_copy(rows_v.at[i], acc_sh.at[dst_v.at[cb + i]],
                                      ssems.at[i]).wait()

                @pl.when(cb + NBUF + i < seg_chunks)
                def _():
                    _gather(cb + NBUF + i, i)
            return 0

        lax.fori_loop(0, seg_chunks // NBUF, step_body, 0)
        return 0

    scope_main = jax.named_scope("ph_main")
    scope_main.__enter__()
    lax.fori_loop(0, 10, seg_body, 0)
    scope_main.__exit__(None, None, None)

    plsc.subcore_barrier()

    scope_dump = jax.named_scope("ph_dump")
    scope_dump.__enter__()

    @pl.when(sub < NUM_SUBCORES - 1)
    def _():
        pltpu.sync_copy(acc_sh.at[pl.ds(sub * STRIPE, STRIPE)],
                        part_hbm.at[core, pl.ds(sub * STRIPE, STRIPE)])

    @pl.when(sub == NUM_SUBCORES - 1)
    def _():
        pltpu.sync_copy(acc_sh.at[pl.ds((NUM_SUBCORES - 1) * STRIPE, LAST_STRIPE)],
                        part_hbm.at[core, pl.ds((NUM_SUBCORES - 1) * STRIPE, LAST_STRIPE)])

    scope_dump.__exit__(None, None, None)


_layer_call = pl.kernel(
    _layer_body,
    out_type=jax.ShapeDtypeStruct((NUM_CORES, N_NODES, DIM), jnp.float32),
    mesh=_MESH,
    scratch_types=[
        pltpu.VMEM((CHUNKS_PER_TILE // 10, CHUNK), jnp.int32),
        pltpu.VMEM((CHUNKS_PER_TILE // 10, CHUNK), jnp.int32),
        pltpu.VMEM((CHUNKS_PER_TILE // 10, CHUNK), jnp.float32),
        pltpu.VMEM((NBUF, CHUNK, DIM), jnp.float32),
        pltpu.VMEM_SHARED((N_NODES, DIM), jnp.float32),
        pltpu.SemaphoreType.DMA((NBUF,)),
        pltpu.SemaphoreType.DMA((NBUF,)),
    ],
    name="lgcn_layer_sc",
)

B_PER_TILE = BATCH // NW


def _score_body(light_hbm, u_hbm, p_hbm, n_hbm, pos_out, neg_out,
                ui_v, pi_v, ni_v, ur_v, pr_v, nr_v, ps_v, ns_v, sem):
    core = lax.axis_index("c")
    sub = lax.axis_index("s")
    wid = core * NUM_SUBCORES + sub
    base = wid * B_PER_TILE

    pltpu.sync_copy(u_hbm.at[pl.ds(base, B_PER_TILE)], ui_v)
    pltpu.sync_copy(p_hbm.at[pl.ds(base, B_PER_TILE)], pi_v)
    pltpu.sync_copy(n_hbm.at[pl.ds(base, B_PER_TILE)], ni_v)

    pltpu.async_copy(light_hbm.at[ui_v], ur_v, sem).wait()
    pltpu.async_copy(light_hbm.at[pi_v], pr_v, sem).wait()
    pltpu.async_copy(light_hbm.at[ni_v], nr_v, sem).wait()

    lane = lax.iota(jnp.int32, 16)

    _dnums = lax.GatherDimensionNumbers(
        offset_dims=(), collapsed_slice_dims=(0,), start_index_map=(0,))

    def _perm(x, idx):
        return lax.gather(x, idx[:, None], _dnums, (1,),
                          mode=lax.GatherScatterMode.PROMISE_IN_BOUNDS)

    def _lane_sum(x):
        for sh in (8, 4, 2, 1):
            x = x + _perm(x, lane ^ sh)
        return x

    def dot_body(g, _):
        pscore = jnp.zeros((16,), jnp.float32)
        nscore = jnp.zeros((16,), jnp.float32)
        for j in range(16):
            i = g * 16 + j
            accp = jnp.zeros((16,), jnp.float32)
            accn = jnp.zeros((16,), jnp.float32)
            for v in range(DIM // 16):
                sl = pl.ds(v * 16, 16)
                uv = ur_v[i, sl]
                accp = accp + uv * pr_v[i, sl]
                accn = accn + uv * nr_v[i, sl]
            pscore = jnp.where(lane == j, _lane_sum(accp), pscore)
            nscore = jnp.where(lane == j, _lane_sum(accn), nscore)
        ps_v[pl.ds(g * 16, 16)] = pscore
        ns_v[pl.ds(g * 16, 16)] = nscore
        return 0

    lax.fori_loop(0, B_PER_TILE // 16, dot_body, 0)

    pltpu.sync_copy(ps_v, pos_out.at[pl.ds(base, B_PER_TILE)])
    pltpu.sync_copy(ns_v, neg_out.at[pl.ds(base, B_PER_TILE)])


_score_call = pl.kernel(
    _score_body,
    out_type=(jax.ShapeDtypeStruct((BATCH,), jnp.float32),
              jax.ShapeDtypeStruct((BATCH,), jnp.float32)),
    mesh=_MESH,
    scratch_types=[
        pltpu.VMEM((B_PER_TILE,), jnp.int32),
        pltpu.VMEM((B_PER_TILE,), jnp.int32),
        pltpu.VMEM((B_PER_TILE,), jnp.int32),
        pltpu.VMEM((B_PER_TILE, DIM), jnp.float32),
        pltpu.VMEM((B_PER_TILE, DIM), jnp.float32),
        pltpu.VMEM((B_PER_TILE, DIM), jnp.float32),
        pltpu.VMEM((B_PER_TILE,), jnp.float32),
        pltpu.VMEM((B_PER_TILE,), jnp.float32),
        pltpu.SemaphoreType.DMA,
    ],
    name="lgcn_score_sc",
)

_ROW_BLK = 1000


def _add2_body(a_ref, b_ref, o_ref):
    o_ref[...] = a_ref[...] + b_ref[...]


def _combine2(a, b):
    spec = pl.BlockSpec((_ROW_BLK, DIM), lambda i: (i, 0))
    return pl.pallas_call(
        _add2_body,
        out_shape=jax.ShapeDtypeStruct((N_NODES, DIM), jnp.float32),
        grid=(N_NODES // _ROW_BLK,),
        in_specs=[spec, spec],
        out_specs=spec,
        name="lgcn_add2_tc",
    )(a, b)


def _final_body(a_ref, b_ref, c_ref, d_ref, o_ref):
    o_ref[...] = (a_ref[...] + b_ref[...] + c_ref[...] + d_ref[...]) * jnp.float32(1.0 / (N_LAYERS + 1))


def _final_combine(a, b, c, d):
    spec = pl.BlockSpec((_ROW_BLK, DIM), lambda i: (i, 0))
    return pl.pallas_call(
        _final_body,
        out_shape=jax.ShapeDtypeStruct((N_NODES, DIM), jnp.float32),
        grid=(N_NODES // _ROW_BLK,),
        in_specs=[spec, spec, spec, spec],
        out_specs=spec,
        name="lgcn_final_tc",
    )(a, b, c, d)


@jax.jit
def kernel(user_nodes, pos_item_nodes, neg_item_nodes, edge_index, edge_weight, emb_user, emb_item):
    e0 = jnp.concatenate([emb_user, emb_item], axis=0)
    pad = NW * CHUNKS_PER_TILE * CHUNK - N_EDGES
    pad_idx = jnp.arange(pad, dtype=jnp.int32) % N_NODES
    src = jnp.concatenate([edge_index[0].astype(jnp.int32), pad_idx])
    dst = jnp.concatenate([edge_index[1].astype(jnp.int32), pad_idx])
    w = jnp.concatenate([edge_weight, jnp.zeros((pad,), jnp.float32)])
    n2d = NW * CHUNKS_PER_TILE
    src = src.reshape(n2d, CHUNK)
    dst = dst.reshape(n2d, CHUNK)
    w = w.reshape(n2d, CHUNK)

    p1 = _layer_call(e0, src, dst, w)
    e1 = _combine2(p1[0], p1[1])
    p2 = _layer_call(e1, src, dst, w)
    light = _final_combine(e0, e1, p2[0], p2[1])

    pos_scores, neg_scores = _score_call(
        light,
        user_nodes.astype(jnp.int32),
        pos_item_nodes.astype(jnp.int32),
        neg_item_nodes.astype(jnp.int32),
    )
    return (pos_scores, neg_scores)


# speedup vs baseline: 1.0149x; 1.0149x over previous
import jax
import jax.numpy as jnp
from jax import lax
from jax.experimental import pallas as pl
from jax.experimental.pallas import tpu as pltpu
from jax.experimental.pallas import tpu_sc as plsc

N_NODES = 10000
DIM = 128
N_EDGES = 320000
BATCH = 4096
N_LAYERS = 2

NUM_CORES = 2
NUM_SUBCORES = 16
NW = NUM_CORES * NUM_SUBCORES

CHUNK = 32
CHUNKS_PER_TILE = 320
NBUF = 4

STRIPE = 632
LAST_STRIPE = N_NODES - (NUM_SUBCORES - 1) * STRIPE

_MESH = plsc.VectorSubcoreMesh(core_axis_name="c", subcore_axis_name="s")


def _make_layer_body(with_init):
    def _layer_body(table_hbm, init0_hbm, init1_hbm, src_hbm, dst_hbm, w_hbm,
                    part_hbm, src_v, dst_v, w_v, rows_v, acc_sh, gsems, ssems):
        core = lax.axis_index("c")
        sub = lax.axis_index("s")
        wid = core * NUM_SUBCORES + sub

        if with_init:
            def _init_from(t_hbm):
                @pl.when(sub < NUM_SUBCORES - 1)
                def _():
                    pltpu.sync_copy(t_hbm.at[pl.ds(sub * STRIPE, STRIPE)],
                                    acc_sh.at[pl.ds(sub * STRIPE, STRIPE)])

                @pl.when(sub == NUM_SUBCORES - 1)
                def _():
                    pltpu.sync_copy(
                        t_hbm.at[pl.ds((NUM_SUBCORES - 1) * STRIPE, LAST_STRIPE)],
                        acc_sh.at[pl.ds((NUM_SUBCORES - 1) * STRIPE, LAST_STRIPE)])

            @pl.when(core == 0)
            def _():
                _init_from(init0_hbm)

            @pl.when(core == 1)
            def _():
                _init_from(init1_hbm)
        else:
            zvec = jnp.zeros((16,), jnp.float32)

            def zfill(r, _):
                for v in range(DIM // 16):
                    rows_v[0, r, pl.ds(v * 16, 16)] = zvec
                return 0

            lax.fori_loop(0, CHUNK, zfill, 0)

            def zcopy(i, _):
                off = jnp.minimum(sub * STRIPE + CHUNK * i, N_NODES - CHUNK)
                pltpu.sync_copy(rows_v.at[0], acc_sh.at[pl.ds(off, CHUNK)])
                return 0

            lax.fori_loop(0, STRIPE // CHUNK + 1, zcopy, 0)

        plsc.subcore_barrier()

        def _scale(b, c):
            def scale_body(g, _):
                wvec = w_v[c, pl.ds(g * 16, 16)]
                for j in range(16):
                    ws = wvec[j]
                    for v in range(DIM // 16):
                        sl = pl.ds(v * 16, 16)
                        rows_v[b, g * 16 + j, sl] = rows_v[b, g * 16 + j, sl] * ws
                return 0

            lax.fori_loop(0, CHUNK // 16, scale_body, 0)

        seg_chunks = CHUNKS_PER_TILE // 8

        def _gather(c, i):
            pltpu.async_copy(table_hbm.at[src_v.at[c]], rows_v.at[i], gsems.at[i])

        def _gather_wait(c, i):
            pltpu.make_async_copy(table_hbm.at[src_v.at[c]], rows_v.at[i],
                                  gsems.at[i]).wait()

        def seg_body(seg, _):
            base_row = wid * CHUNKS_PER_TILE + seg * seg_chunks
            pltpu.sync_copy(src_hbm.at[pl.ds(base_row, seg_chunks)], src_v)
            pltpu.sync_copy(dst_hbm.at[pl.ds(base_row, seg_chunks)], dst_v)
            pltpu.sync_copy(w_hbm.at[pl.ds(base_row, seg_chunks)], w_v)

            for i in range(NBUF):
                _gather(i, i)

            def step_body(q, _):
                cb = q * NBUF
                for i in range(NBUF):
                    _gather_wait(cb + i, i)
                    _scale(i, cb + i)
                    pltpu.async_copy(rows_v.at[i], acc_sh.at[dst_v.at[cb + i]],
                                     ssems.at[i], add=True)
                for i in range(NBUF):
                    pltpu.make_async_copy(rows_v.at[i], acc_sh.at[dst_v.at[cb + i]],
                                          ssems.at[i]).wait()

                    @pl.when(cb + NBUF + i < seg_chunks)
                    def _():
                        _gather(cb + NBUF + i, i)
                return 0

            lax.fori_loop(0, seg_chunks // NBUF, step_body, 0)
            return 0

        lax.fori_loop(0, 8, seg_body, 0)

        plsc.subcore_barrier()

        @pl.when(sub < NUM_SUBCORES - 1)
        def _():
            pltpu.sync_copy(acc_sh.at[pl.ds(sub * STRIPE, STRIPE)],
                            part_hbm.at[core, pl.ds(sub * STRIPE, STRIPE)])

        @pl.when(sub == NUM_SUBCORES - 1)
        def _():
            pltpu.sync_copy(
                acc_sh.at[pl.ds((NUM_SUBCORES - 1) * STRIPE, LAST_STRIPE)],
                part_hbm.at[core, pl.ds((NUM_SUBCORES - 1) * STRIPE, LAST_STRIPE)])

    return _layer_body


def _make_layer_call(with_init, name):
    return pl.kernel(
        _make_layer_body(with_init),
        out_type=jax.ShapeDtypeStruct((NUM_CORES, N_NODES, DIM), jnp.float32),
        mesh=_MESH,
        scratch_types=[
            pltpu.VMEM((CHUNKS_PER_TILE // 8, CHUNK), jnp.int32),
            pltpu.VMEM((CHUNKS_PER_TILE // 8, CHUNK), jnp.int32),
            pltpu.VMEM((CHUNKS_PER_TILE // 8, CHUNK), jnp.float32),
            pltpu.VMEM((NBUF, CHUNK, DIM), jnp.float32),
            pltpu.VMEM_SHARED((N_NODES, DIM), jnp.float32),
            pltpu.SemaphoreType.DMA((NBUF,)),
            pltpu.SemaphoreType.DMA((NBUF,)),
        ],
        name=name,
    )


_layer1_call = _make_layer_call(False, "lgcn_layer1_sc")
_layer2_call = _make_layer_call(True, "lgcn_layer2_sc")

B_PER_TILE = BATCH // NW


def _score_body(t0_hbm, t1_hbm, u_hbm, p_hbm, n_hbm, pos_out, neg_out,
                ui_v, pi_v, ni_v, u0_v, p0_v, n0_v, u1_v, p1_v, n1_v,
                ps_v, ns_v, sem):
    core = lax.axis_index("c")
    sub = lax.axis_index("s")
    wid = core * NUM_SUBCORES + sub
    base = wid * B_PER_TILE

    pltpu.sync_copy(u_hbm.at[pl.ds(base, B_PER_TILE)], ui_v)
    pltpu.sync_copy(p_hbm.at[pl.ds(base, B_PER_TILE)], pi_v)
    pltpu.sync_copy(n_hbm.at[pl.ds(base, B_PER_TILE)], ni_v)

    pltpu.async_copy(t0_hbm.at[ui_v], u0_v, sem).wait()
    pltpu.async_copy(t0_hbm.at[pi_v], p0_v, sem).wait()
    pltpu.async_copy(t0_hbm.at[ni_v], n0_v, sem).wait()
    pltpu.async_copy(t1_hbm.at[ui_v], u1_v, sem).wait()
    pltpu.async_copy(t1_hbm.at[pi_v], p1_v, sem).wait()
    pltpu.async_copy(t1_hbm.at[ni_v], n1_v, sem).wait()

    lane = lax.iota(jnp.int32, 16)
    ninth = jnp.float32(1.0 / ((N_LAYERS + 1) * (N_LAYERS + 1)))

    _dnums = lax.GatherDimensionNumbers(
        offset_dims=(), collapsed_slice_dims=(0,), start_index_map=(0,))

    def _perm(x, idx):
        return lax.gather(x, idx[:, None], _dnums, (1,),
                          mode=lax.GatherScatterMode.PROMISE_IN_BOUNDS)

    def _lane_sum(x):
        for sh in (8, 4, 2, 1):
            x = x + _perm(x, lane ^ sh)
        return x

    def dot_body(g, _):
        pscore = jnp.zeros((16,), jnp.float32)
        nscore = jnp.zeros((16,), jnp.float32)
        for j in range(16):
            i = g * 16 + j
            accp = jnp.zeros((16,), jnp.float32)
            accn = jnp.zeros((16,), jnp.float32)
            for v in range(DIM // 16):
                sl = pl.ds(v * 16, 16)
                uv = u0_v[i, sl] + u1_v[i, sl]
                accp = accp + uv * (p0_v[i, sl] + p1_v[i, sl])
                accn = accn + uv * (n0_v[i, sl] + n1_v[i, sl])
            pscore = jnp.where(lane == j, _lane_sum(accp), pscore)
            nscore = jnp.where(lane == j, _lane_sum(accn), nscore)
        ps_v[pl.ds(g * 16, 16)] = pscore * ninth
        ns_v[pl.ds(g * 16, 16)] = nscore * ninth
        return 0

    lax.fori_loop(0, B_PER_TILE // 16, dot_body, 0)

    pltpu.sync_copy(ps_v, pos_out.at[pl.ds(base, B_PER_TILE)])
    pltpu.sync_copy(ns_v, neg_out.at[pl.ds(base, B_PER_TILE)])


_score_call = pl.kernel(
    _score_body,
    out_type=(jax.ShapeDtypeStruct((BATCH,), jnp.float32),
              jax.ShapeDtypeStruct((BATCH,), jnp.float32)),
    mesh=_MESH,
    scratch_types=[
        pltpu.VMEM((B_PER_TILE,), jnp.int32),
        pltpu.VMEM((B_PER_TILE,), jnp.int32),
        pltpu.VMEM((B_PER_TILE,), jnp.int32),
        pltpu.VMEM((B_PER_TILE, DIM), jnp.float32),
        pltpu.VMEM((B_PER_TILE, DIM), jnp.float32),
        pltpu.VMEM((B_PER_TILE, DIM), jnp.float32),
        pltpu.VMEM((B_PER_TILE, DIM), jnp.float32),
        pltpu.VMEM((B_PER_TILE, DIM), jnp.float32),
        pltpu.VMEM((B_PER_TILE, DIM), jnp.float32),
        pltpu.VMEM((B_PER_TILE,), jnp.float32),
        pltpu.VMEM((B_PER_TILE,), jnp.float32),
        pltpu.SemaphoreType.DMA,
    ],
    name="lgcn_score_sc",
)

_ROW_BLK = 1000


def _add2_body(a_ref, b_ref, o_ref):
    o_ref[...] = a_ref[...] + b_ref[...]


def _combine2(a, b):
    spec = pl.BlockSpec((_ROW_BLK, DIM), lambda i: (i, 0))
    return pl.pallas_call(
        _add2_body,
        out_shape=jax.ShapeDtypeStruct((N_NODES, DIM), jnp.float32),
        grid=(N_NODES // _ROW_BLK,),
        in_specs=[spec, spec],
        out_specs=spec,
        name="lgcn_add2_tc",
    )(a, b)


@jax.jit
def kernel(user_nodes, pos_item_nodes, neg_item_nodes, edge_index, edge_weight, emb_user, emb_item):
    e0 = jnp.concatenate([emb_user, emb_item], axis=0)
    pad = NW * CHUNKS_PER_TILE * CHUNK - N_EDGES
    pad_idx = jnp.arange(pad, dtype=jnp.int32) % N_NODES
    src = jnp.concatenate([edge_index[0].astype(jnp.int32), pad_idx])
    dst = jnp.concatenate([edge_index[1].astype(jnp.int32), pad_idx])
    w = jnp.concatenate([edge_weight, jnp.zeros((pad,), jnp.float32)])
    n2d = NW * CHUNKS_PER_TILE
    src = src.reshape(n2d, CHUNK)
    dst = dst.reshape(n2d, CHUNK)
    w = w.reshape(n2d, CHUNK)

    p1 = _layer1_call(e0, e0, e0, src, dst, w)
    e1 = _combine2(p1[0], p1[1])
    p2 = _layer2_call(e1, e0, e1, src, dst, w)

    pos_scores, neg_scores = _score_call(
        p2[0], p2[1],
        user_nodes.astype(jnp.int32),
        pos_item_nodes.astype(jnp.int32),
        neg_item_nodes.astype(jnp.int32),
    )
    return (pos_scores, neg_scores)


# speedup vs baseline: 1.0480x; 1.0326x over previous
import jax
import jax.numpy as jnp
from jax import lax
from jax.experimental import pallas as pl
from jax.experimental.pallas import tpu as pltpu
from jax.experimental.pallas import tpu_sc as plsc

N_NODES = 10000
DIM = 128
N_EDGES = 320000
BATCH = 4096
N_LAYERS = 2

NUM_CORES = 2
NUM_SUBCORES = 16
NW = NUM_CORES * NUM_SUBCORES

CHUNK = 32
CHUNKS_PER_TILE = 320
NBUF = 4

STRIPE = 632
LAST_STRIPE = N_NODES - (NUM_SUBCORES - 1) * STRIPE

_MESH = plsc.VectorSubcoreMesh(core_axis_name="c", subcore_axis_name="s")


def _layer_body(table_hbm, src_hbm, dst_hbm, w_hbm, part_hbm,
                src_v, dst_v, w_v, rows_v, acc_sh, gsems, ssems):
    core = lax.axis_index("c")
    sub = lax.axis_index("s")
    wid = core * NUM_SUBCORES + sub

    zvec = jnp.zeros((16,), jnp.float32)

    def zfill(r, _):
        for v in range(DIM // 16):
            rows_v[0, r, pl.ds(v * 16, 16)] = zvec
        return 0

    lax.fori_loop(0, CHUNK, zfill, 0)

    def zcopy(i, _):
        off = jnp.minimum(sub * STRIPE + CHUNK * i, N_NODES - CHUNK)
        pltpu.sync_copy(rows_v.at[0], acc_sh.at[pl.ds(off, CHUNK)])
        return 0

    lax.fori_loop(0, STRIPE // CHUNK + 1, zcopy, 0)

    plsc.subcore_barrier()

    def _scale(b, c):
        def scale_body(g, _):
            wvec = w_v[c, pl.ds(g * 16, 16)]
            for j in range(16):
                ws = wvec[j]
                for v in range(DIM // 16):
                    sl = pl.ds(v * 16, 16)
                    rows_v[b, g * 16 + j, sl] = rows_v[b, g * 16 + j, sl] * ws
            return 0

        lax.fori_loop(0, CHUNK // 16, scale_body, 0)

    seg_chunks = CHUNKS_PER_TILE // 8

    def _gather(c, i):
        pltpu.async_copy(table_hbm.at[src_v.at[c]], rows_v.at[i], gsems.at[i])

    def _gather_wait(c, i):
        pltpu.make_async_copy(table_hbm.at[src_v.at[c]], rows_v.at[i],
                              gsems.at[i]).wait()

    def seg_body(seg, _):
        base_row = wid * CHUNKS_PER_TILE + seg * seg_chunks
        pltpu.sync_copy(src_hbm.at[pl.ds(base_row, seg_chunks)], src_v)
        pltpu.sync_copy(dst_hbm.at[pl.ds(base_row, seg_chunks)], dst_v)
        pltpu.sync_copy(w_hbm.at[pl.ds(base_row, seg_chunks)], w_v)

        for i in range(NBUF):
            _gather(i, i)

        def step_body(q, _):
            cb = q * NBUF
            for i in range(NBUF):
                _gather_wait(cb + i, i)
                _scale(i, cb + i)
                pltpu.async_copy(rows_v.at[i], acc_sh.at[dst_v.at[cb + i]],
                                 ssems.at[i], add=True)
            for i in range(NBUF):
                pltpu.make_async_copy(rows_v.at[i], acc_sh.at[dst_v.at[cb + i]],
                                      ssems.at[i]).wait()

                @pl.when(cb + NBUF + i < seg_chunks)
                def _():
                    _gather(cb + NBUF + i, i)
            return 0

        lax.fori_loop(0, seg_chunks // NBUF, step_body, 0)
        return 0

    lax.fori_loop(0, 8, seg_body, 0)

    plsc.subcore_barrier()

    @pl.when(sub < NUM_SUBCORES - 1)
    def _():
        pltpu.sync_copy(acc_sh.at[pl.ds(sub * STRIPE, STRIPE)],
                        part_hbm.at[core, pl.ds(sub * STRIPE, STRIPE)])

    @pl.when(sub == NUM_SUBCORES - 1)
    def _():
        pltpu.sync_copy(acc_sh.at[pl.ds((NUM_SUBCORES - 1) * STRIPE, LAST_STRIPE)],
                        part_hbm.at[core, pl.ds((NUM_SUBCORES - 1) * STRIPE, LAST_STRIPE)])


_layer_call = pl.kernel(
    _layer_body,
    out_type=jax.ShapeDtypeStruct((NUM_CORES, N_NODES, DIM), jnp.float32),
    mesh=_MESH,
    scratch_types=[
        pltpu.VMEM((CHUNKS_PER_TILE // 8, CHUNK), jnp.int32),
        pltpu.VMEM((CHUNKS_PER_TILE // 8, CHUNK), jnp.int32),
        pltpu.VMEM((CHUNKS_PER_TILE // 8, CHUNK), jnp.float32),
        pltpu.VMEM((NBUF, CHUNK, DIM), jnp.float32),
        pltpu.VMEM_SHARED((N_NODES, DIM), jnp.float32),
        pltpu.SemaphoreType.DMA((NBUF,)),
        pltpu.SemaphoreType.DMA((NBUF,)),
    ],
    name="lgcn_layer_sc",
)

B_PER_TILE = BATCH // NW


def _score_body(light_hbm, u_hbm, p_hbm, n_hbm, pos_out, neg_out,
                ui_v, pi_v, ni_v, ur_v, pr_v, nr_v, ps_v, ns_v, sem):
    core = lax.axis_index("c")
    sub = lax.axis_index("s")
    wid = core * NUM_SUBCORES + sub
    base = wid * B_PER_TILE

    pltpu.sync_copy(u_hbm.at[pl.ds(base, B_PER_TILE)], ui_v)
    pltpu.sync_copy(p_hbm.at[pl.ds(base, B_PER_TILE)], pi_v)
    pltpu.sync_copy(n_hbm.at[pl.ds(base, B_PER_TILE)], ni_v)

    pltpu.async_copy(light_hbm.at[ui_v], ur_v, sem).wait()
    pltpu.async_copy(light_hbm.at[pi_v], pr_v, sem).wait()
    pltpu.async_copy(light_hbm.at[ni_v], nr_v, sem).wait()

    lane = lax.iota(jnp.int32, 16)

    _dnums = lax.GatherDimensionNumbers(
        offset_dims=(), collapsed_slice_dims=(0,), start_index_map=(0,))

    def _perm(x, idx):
        return lax.gather(x, idx[:, None], _dnums, (1,),
                          mode=lax.GatherScatterMode.PROMISE_IN_BOUNDS)

    def _lane_sum(x):
        for sh in (8, 4, 2, 1):
            x = x + _perm(x, lane ^ sh)
        return x

    def dot_body(g, _):
        pscore = jnp.zeros((16,), jnp.float32)
        nscore = jnp.zeros((16,), jnp.float32)
        for j in range(16):
            i = g * 16 + j
            accp = jnp.zeros((16,), jnp.float32)
            accn = jnp.zeros((16,), jnp.float32)
            for v in range(DIM // 16):
                sl = pl.ds(v * 16, 16)
                uv = ur_v[i, sl]
                accp = accp + uv * pr_v[i, sl]
                accn = accn + uv * nr_v[i, sl]
            pscore = jnp.where(lane == j, _lane_sum(accp), pscore)
            nscore = jnp.where(lane == j, _lane_sum(accn), nscore)
        ps_v[pl.ds(g * 16, 16)] = pscore
        ns_v[pl.ds(g * 16, 16)] = nscore
        return 0

    lax.fori_loop(0, B_PER_TILE // 16, dot_body, 0)

    pltpu.sync_copy(ps_v, pos_out.at[pl.ds(base, B_PER_TILE)])
    pltpu.sync_copy(ns_v, neg_out.at[pl.ds(base, B_PER_TILE)])


_score_call = pl.kernel(
    _score_body,
    out_type=(jax.ShapeDtypeStruct((BATCH,), jnp.float32),
              jax.ShapeDtypeStruct((BATCH,), jnp.float32)),
    mesh=_MESH,
    scratch_types=[
        pltpu.VMEM((B_PER_TILE,), jnp.int32),
        pltpu.VMEM((B_PER_TILE,), jnp.int32),
        pltpu.VMEM((B_PER_TILE,), jnp.int32),
        pltpu.VMEM((B_PER_TILE, DIM), jnp.float32),
        pltpu.VMEM((B_PER_TILE, DIM), jnp.float32),
        pltpu.VMEM((B_PER_TILE, DIM), jnp.float32),
        pltpu.VMEM((B_PER_TILE,), jnp.float32),
        pltpu.VMEM((B_PER_TILE,), jnp.float32),
        pltpu.SemaphoreType.DMA,
    ],
    name="lgcn_score_sc",
)

_ROW_BLK = 1000


def _add2_body(a_ref, b_ref, o_ref):
    o_ref[...] = a_ref[...] + b_ref[...]


def _combine2(a, b):
    spec = pl.BlockSpec((_ROW_BLK, DIM), lambda i: (i, 0))
    return pl.pallas_call(
        _add2_body,
        out_shape=jax.ShapeDtypeStruct((N_NODES, DIM), jnp.float32),
        grid=(N_NODES // _ROW_BLK,),
        in_specs=[spec, spec],
        out_specs=spec,
        name="lgcn_add2_tc",
    )(a, b)


def _final_body(a_ref, b_ref, c_ref, d_ref, o_ref):
    o_ref[...] = (a_ref[...] + b_ref[...] + c_ref[...] + d_ref[...]) * jnp.float32(1.0 / (N_LAYERS + 1))


def _final_combine(a, b, c, d):
    spec = pl.BlockSpec((_ROW_BLK, DIM), lambda i: (i, 0))
    return pl.pallas_call(
        _final_body,
        out_shape=jax.ShapeDtypeStruct((N_NODES, DIM), jnp.float32),
        grid=(N_NODES // _ROW_BLK,),
        in_specs=[spec, spec, spec, spec],
        out_specs=spec,
        name="lgcn_final_tc",
    )(a, b, c, d)


@jax.jit
def kernel(user_nodes, pos_item_nodes, neg_item_nodes, edge_index, edge_weight, emb_user, emb_item):
    e0 = jnp.concatenate([emb_user, emb_item], axis=0)
    pad = NW * CHUNKS_PER_TILE * CHUNK - N_EDGES
    pad_idx = jnp.arange(pad, dtype=jnp.int32) % N_NODES
    src = jnp.concatenate([edge_index[0].astype(jnp.int32), pad_idx])
    dst = jnp.concatenate([edge_index[1].astype(jnp.int32), pad_idx])
    w = jnp.concatenate([edge_weight, jnp.zeros((pad,), jnp.float32)])
    n2d = NW * CHUNKS_PER_TILE
    src = src.reshape(n2d, CHUNK)
    dst = dst.reshape(n2d, CHUNK)
    w = w.reshape(n2d, CHUNK)

    p1 = _layer_call(e0, src, dst, w)
    e1 = _combine2(p1[0], p1[1])
    p2 = _layer_call(e1, src, dst, w)
    light = _final_combine(e0, e1, p2[0], p2[1])

    pos_scores, neg_scores = _score_call(
        light,
        user_nodes.astype(jnp.int32),
        pos_item_nodes.astype(jnp.int32),
        neg_item_nodes.astype(jnp.int32),
    )
    return (pos_scores, neg_scores)
